# baseline (speedup 1.0000x reference)
import functools

import jax
import jax.numpy as jnp
from jax import lax
from jax.experimental import pallas as pl
from jax.experimental.pallas import tpu as pltpu
from jax.experimental.pallas import tpu_sc as plsc

_HID = 32
_CB = 8192
_IN = 256
_N = 8192
_TB = 256
_NBLK = _N // _TB


_BIG = 2**30


def _dist_argmin_body(x_ref, wp_ref, bp_ref, cb_ref, idx_ref, mv_ref):
    h = jnp.dot(x_ref[...], wp_ref[...]) + bp_ref[...][None, :]
    mm = jnp.dot(h, cb_ref[...])
    hnorm = jnp.sum(h * h, axis=1, keepdims=True)
    enorm = jnp.sum(cb_ref[...] * cb_ref[...], axis=0, keepdims=True)
    dist = hnorm - 2.0 * mm + enorm

    m = jnp.min(dist, axis=1, keepdims=True)
    ebits = lax.bitcast_convert_type(m, jnp.int32) & jnp.int32(0x7F800000)
    ulp = lax.bitcast_convert_type(ebits, jnp.float32) * jnp.float32(2.0**-7)
    zone = dist <= m + 4.0 * ulp
    iota = lax.broadcasted_iota(jnp.int32, dist.shape, 1)
    cand_iota = jnp.where(zone, iota, _BIG)
    chunk = iota >> 11

    def next_j(av, lastj):
        avq = av.astype(jnp.bfloat16).astype(jnp.float32)
        same = chunk == (lastj >> 11)
        thr = jnp.where(same, av, avq)
        elig = (iota > lastj) & (dist < thr)
        return jnp.min(jnp.where(elig, cand_iota, _BIG), axis=1, keepdims=True)

    def chain_body(carry):
        av, ai, lastj, jn = carry
        upd = jn < _BIG
        x = jnp.min(jnp.where(iota == jn, dist, jnp.inf), axis=1, keepdims=True)
        av = jnp.where(upd, x, av)
        ai = jnp.where(upd, jn, ai)
        lastj = jnp.where(upd, jn, lastj)
        return av, ai, lastj, next_j(av, lastj)

    def chain_cond(carry):
        return jnp.any(carry[3] < _BIG)

    shape = (dist.shape[0], 1)
    init_av = jnp.full(shape, jnp.inf, jnp.float32)
    init_last = jnp.full(shape, -1, jnp.int32)
    init = (
        init_av,
        jnp.zeros(shape, jnp.int32),
        init_last,
        jnp.min(cand_iota, axis=1, keepdims=True),
    )
    av, ai, _, _ = lax.while_loop(chain_cond, chain_body, init)
    idx_ref[0, 0, :] = ai[:, 0]
    mv_ref[0, 0, :] = av[:, 0]


def _out_table_body(cb_ref, wb_ref, bb_ref, mv_ref, tab_ref, lsum_ref):
    i = pl.program_id(0)
    cb = cb_ref[...]
    tab_ref[...] = lax.dot_general(
        cb, wb_ref[...], (((0,), (0,)), ((), ()))
    ) + bb_ref[...][None, :]
    blk = jnp.sum(mv_ref[...]).reshape(1, 1)

    @pl.when(i == 0)
    def _init():
        lsum_ref[...] = blk

    @pl.when(i > 0)
    def _acc():
        lsum_ref[...] += blk


def _make_sc_gather(n_rows, row_dim, chunk=128):
    info = plsc.get_sparse_core_info()
    nc, ns = info.num_cores, info.num_subcores
    nw = nc * ns
    b_per_w = n_rows // nw
    mesh = plsc.VectorSubcoreMesh(core_axis_name="c", subcore_axis_name="s")

    @functools.partial(
        pl.kernel, mesh=mesh,
        out_type=jax.ShapeDtypeStruct((n_rows, row_dim), jnp.float32),
        scratch_types=[
            pltpu.VMEM((b_per_w,), jnp.int32),
            pltpu.VMEM((b_per_w, row_dim), jnp.float32),
            pltpu.SemaphoreType.DMA,
        ],
    )
    def gather_k(idx_hbm, table_hbm, out_hbm, idx_v, rows_v, sem):
        wid = lax.axis_index("s") * nc + lax.axis_index("c")
        base = wid * b_per_w
        pltpu.sync_copy(idx_hbm.at[pl.ds(base, b_per_w)], idx_v)
        handles = [
            pltpu.async_copy(
                table_hbm.at[idx_v.at[pl.ds(j * chunk, chunk)]],
                rows_v.at[pl.ds(j * chunk, chunk)],
                sem,
            )
            for j in range(b_per_w // chunk)
        ]
        for h in handles:
            h.wait()
        pltpu.sync_copy(rows_v, out_hbm.at[pl.ds(base, b_per_w)])

    return gather_k


def kernel(state, W_proj, b_proj, W_back, b_back, codebook):
    bsz, T, _ = state.shape
    x = state.reshape(_N, _IN)

    idx3, mv3 = pl.pallas_call(
        _dist_argmin_body,
        grid=(_NBLK,),
        in_specs=[
            pl.BlockSpec((_TB, _IN), lambda i: (i, 0)),
            pl.BlockSpec((_IN, _HID), lambda i: (0, 0)),
            pl.BlockSpec((_HID,), lambda i: (0,)),
            pl.BlockSpec((_HID, _CB), lambda i: (0, 0)),
        ],
        out_specs=[
            pl.BlockSpec((1, 1, _TB), lambda i: (i, 0, 0)),
            pl.BlockSpec((1, 1, _TB), lambda i: (i, 0, 0)),
        ],
        out_shape=[
            jax.ShapeDtypeStruct((_NBLK, 1, _TB), jnp.int32),
            jax.ShapeDtypeStruct((_NBLK, 1, _TB), jnp.float32),
        ],
    )(x, W_proj, b_proj, codebook)

    table, lsum = pl.pallas_call(
        _out_table_body,
        grid=(8,),
        in_specs=[
            pl.BlockSpec((_HID, _CB // 8), lambda i: (0, i)),
            pl.BlockSpec((_HID, _IN), lambda i: (0, 0)),
            pl.BlockSpec((_IN,), lambda i: (0,)),
            pl.BlockSpec((_NBLK // 8, 1, _TB), lambda i: (i, 0, 0)),
        ],
        out_specs=[
            pl.BlockSpec((_CB // 8, _IN), lambda i: (i, 0)),
            pl.BlockSpec((1, 1), lambda i: (0, 0)),
        ],
        out_shape=[
            jax.ShapeDtypeStruct((_CB, _IN), jnp.float32),
            jax.ShapeDtypeStruct((1, 1), jnp.float32),
        ],
    )(codebook, W_back, b_back, mv3)

    idx = idx3.reshape(_N)
    out_rows = _make_sc_gather(_N, _IN)(idx, table)

    out = out_rows.reshape(bsz, T, _IN)
    cb_loss = lsum[0, 0] / jnp.float32(_N * _HID)
    att_scores = jnp.zeros((1, 1, 2), dtype=jnp.float32)
    return out, cb_loss, att_scores


# speedup vs baseline: 3.2565x; 3.2565x over previous
import functools

import jax
import jax.numpy as jnp
from jax import lax
from jax.experimental import pallas as pl
from jax.experimental.pallas import tpu as pltpu
from jax.experimental.pallas import tpu_sc as plsc

_HID = 32
_CB = 8192
_IN = 256
_N = 8192
_TB = 256
_NBLK = _N // _TB


_BIG = 2**30
_CHUNK = 2048


def _dist_argmin_body(x_ref, wp_ref, bp_ref, cb_ref, idx_ref, mv_ref):
    h = jnp.dot(x_ref[...], wp_ref[...]) + bp_ref[...][None, :]
    mm = jnp.dot(h, cb_ref[...])
    hnorm = jnp.sum(h * h, axis=1, keepdims=True)
    enorm = jnp.sum(cb_ref[...] * cb_ref[...], axis=0, keepdims=True)
    dist = hnorm - 2.0 * mm + enorm

    iota = lax.broadcasted_iota(jnp.int32, dist.shape, 1)
    av = ai = mval = None
    for c in range(_CB // _CHUNK):
        dc = dist[:, c * _CHUNK:(c + 1) * _CHUNK]
        ic = iota[:, c * _CHUNK:(c + 1) * _CHUNK]
        mc = jnp.min(dc, axis=1, keepdims=True)
        jc = jnp.min(jnp.where(dc == mc, ic, _BIG), axis=1, keepdims=True)
        if av is None:
            av, ai, mval = mc, jc, mc
        else:
            avq = av.astype(jnp.bfloat16).astype(jnp.float32)
            upd = mc < avq
            av = jnp.where(upd, mc, avq)
            ai = jnp.where(upd, jc, ai)
            mval = jnp.where(upd, mc, mval)
    idx_ref[0, 0, :] = ai[:, 0]
    mv_ref[0, 0, :] = mval[:, 0]


def _out_table_body(cb_ref, wb_ref, bb_ref, mv_ref, tab_ref, lsum_ref):
    i = pl.program_id(0)
    cb = cb_ref[...]
    tab_ref[...] = lax.dot_general(
        cb, wb_ref[...], (((0,), (0,)), ((), ()))
    ) + bb_ref[...][None, :]
    blk = jnp.sum(mv_ref[...]).reshape(1, 1)

    @pl.when(i == 0)
    def _init():
        lsum_ref[...] = blk

    @pl.when(i > 0)
    def _acc():
        lsum_ref[...] += blk


def _make_sc_gather(n_rows, row_dim, chunk=128):
    info = plsc.get_sparse_core_info()
    nc, ns = info.num_cores, info.num_subcores
    nw = nc * ns
    b_per_w = n_rows // nw
    mesh = plsc.VectorSubcoreMesh(core_axis_name="c", subcore_axis_name="s")

    @functools.partial(
        pl.kernel, mesh=mesh,
        out_type=jax.ShapeDtypeStruct((n_rows, row_dim), jnp.float32),
        scratch_types=[
            pltpu.VMEM((b_per_w,), jnp.int32),
            pltpu.VMEM((b_per_w, row_dim), jnp.float32),
            pltpu.SemaphoreType.DMA,
        ],
    )
    def gather_k(idx_hbm, table_hbm, out_hbm, idx_v, rows_v, sem):
        wid = lax.axis_index("s") * nc + lax.axis_index("c")
        base = wid * b_per_w
        pltpu.sync_copy(idx_hbm.at[pl.ds(base, b_per_w)], idx_v)
        handles = [
            pltpu.async_copy(
                table_hbm.at[idx_v.at[pl.ds(j * chunk, chunk)]],
                rows_v.at[pl.ds(j * chunk, chunk)],
                sem,
            )
            for j in range(b_per_w // chunk)
        ]
        for h in handles:
            h.wait()
        pltpu.sync_copy(rows_v, out_hbm.at[pl.ds(base, b_per_w)])

    return gather_k


def kernel(state, W_proj, b_proj, W_back, b_back, codebook):
    bsz, T, _ = state.shape
    x = state.reshape(_N, _IN)

    idx3, mv3 = pl.pallas_call(
        _dist_argmin_body,
        grid=(_NBLK,),
        in_specs=[
            pl.BlockSpec((_TB, _IN), lambda i: (i, 0)),
            pl.BlockSpec((_IN, _HID), lambda i: (0, 0)),
            pl.BlockSpec((_HID,), lambda i: (0,)),
            pl.BlockSpec((_HID, _CB), lambda i: (0, 0)),
        ],
        out_specs=[
            pl.BlockSpec((1, 1, _TB), lambda i: (i, 0, 0)),
            pl.BlockSpec((1, 1, _TB), lambda i: (i, 0, 0)),
        ],
        out_shape=[
            jax.ShapeDtypeStruct((_NBLK, 1, _TB), jnp.int32),
            jax.ShapeDtypeStruct((_NBLK, 1, _TB), jnp.float32),
        ],
    )(x, W_proj, b_proj, codebook)

    table, lsum = pl.pallas_call(
        _out_table_body,
        grid=(8,),
        in_specs=[
            pl.BlockSpec((_HID, _CB // 8), lambda i: (0, i)),
            pl.BlockSpec((_HID, _IN), lambda i: (0, 0)),
            pl.BlockSpec((_IN,), lambda i: (0,)),
            pl.BlockSpec((_NBLK // 8, 1, _TB), lambda i: (i, 0, 0)),
        ],
        out_specs=[
            pl.BlockSpec((_CB // 8, _IN), lambda i: (i, 0)),
            pl.BlockSpec((1, 1), lambda i: (0, 0)),
        ],
        out_shape=[
            jax.ShapeDtypeStruct((_CB, _IN), jnp.float32),
            jax.ShapeDtypeStruct((1, 1), jnp.float32),
        ],
    )(codebook, W_back, b_back, mv3)

    idx = idx3.reshape(_N)
    out_rows = _make_sc_gather(_N, _IN)(idx, table)

    out = out_rows.reshape(bsz, T, _IN)
    cb_loss = lsum[0, 0] / jnp.float32(_N * _HID)
    att_scores = jnp.zeros((1, 1, 2), dtype=jnp.float32)
    return out, cb_loss, att_scores


# speedup vs baseline: 3.4361x; 1.0551x over previous
import functools

import jax
import jax.numpy as jnp
from jax import lax
from jax.experimental import pallas as pl
from jax.experimental.pallas import tpu as pltpu
from jax.experimental.pallas import tpu_sc as plsc

_HID = 32
_CB = 8192
_IN = 256
_N = 8192
_TB = 512
_NBLK = _N // _TB


_BIG = 2**30
_CHUNK = 2048


def _dist_argmin_body(x_ref, wp_ref, bp_ref, cb_ref, idx_ref, mv_ref):
    h = jnp.dot(x_ref[...], wp_ref[...]) + bp_ref[...][None, :]
    mm = jnp.dot(h, cb_ref[...])
    hnorm = jnp.sum(h * h, axis=1, keepdims=True)
    enorm = jnp.sum(cb_ref[...] * cb_ref[...], axis=0, keepdims=True)
    dist = hnorm - 2.0 * mm + enorm

    av = ai = mval = None
    for c in range(_CB // _CHUNK):
        dc = dist[:, c * _CHUNK:(c + 1) * _CHUNK]
        mc = jnp.min(dc, axis=1, keepdims=True)
        jc = (jnp.argmin(dc, axis=1).astype(jnp.int32)
              + jnp.int32(c * _CHUNK))[:, None]
        if av is None:
            av, ai, mval = mc, jc, mc
        else:
            avq = av.astype(jnp.bfloat16).astype(jnp.float32)
            upd = mc < avq
            av = jnp.where(upd, mc, avq)
            ai = jnp.where(upd, jc, ai)
            mval = jnp.where(upd, mc, mval)
    idx_ref[0, 0, :] = ai[:, 0]
    mv_ref[0, 0, :] = mval[:, 0]


def _out_table_body(cb_ref, wb_ref, bb_ref, mv_ref, tab_ref, lsum_ref):
    i = pl.program_id(0)
    cb = cb_ref[...]
    tab_ref[...] = lax.dot_general(
        cb, wb_ref[...], (((0,), (0,)), ((), ()))
    ) + bb_ref[...][None, :]
    blk = jnp.sum(mv_ref[...]).reshape(1, 1)

    @pl.when(i == 0)
    def _init():
        lsum_ref[...] = blk

    @pl.when(i > 0)
    def _acc():
        lsum_ref[...] += blk


def _make_sc_gather(n_rows, row_dim, chunk=128):
    info = plsc.get_sparse_core_info()
    nc, ns = info.num_cores, info.num_subcores
    nw = nc * ns
    b_per_w = n_rows // nw
    mesh = plsc.VectorSubcoreMesh(core_axis_name="c", subcore_axis_name="s")

    @functools.partial(
        pl.kernel, mesh=mesh,
        out_type=jax.ShapeDtypeStruct((n_rows, row_dim), jnp.float32),
        scratch_types=[
            pltpu.VMEM((b_per_w,), jnp.int32),
            pltpu.VMEM((b_per_w, row_dim), jnp.float32),
            pltpu.SemaphoreType.DMA,
        ],
    )
    def gather_k(idx_hbm, table_hbm, out_hbm, idx_v, rows_v, sem):
        wid = lax.axis_index("s") * nc + lax.axis_index("c")
        base = wid * b_per_w
        pltpu.sync_copy(idx_hbm.at[pl.ds(base, b_per_w)], idx_v)
        handles = [
            pltpu.async_copy(
                table_hbm.at[idx_v.at[pl.ds(j * chunk, chunk)]],
                rows_v.at[pl.ds(j * chunk, chunk)],
                sem,
            )
            for j in range(b_per_w // chunk)
        ]
        for h in handles:
            h.wait()
        pltpu.sync_copy(rows_v, out_hbm.at[pl.ds(base, b_per_w)])

    return gather_k


def kernel(state, W_proj, b_proj, W_back, b_back, codebook):
    bsz, T, _ = state.shape
    x = state.reshape(_N, _IN)

    idx3, mv3 = pl.pallas_call(
        _dist_argmin_body,
        grid=(_NBLK,),
        in_specs=[
            pl.BlockSpec((_TB, _IN), lambda i: (i, 0)),
            pl.BlockSpec((_IN, _HID), lambda i: (0, 0)),
            pl.BlockSpec((_HID,), lambda i: (0,)),
            pl.BlockSpec((_HID, _CB), lambda i: (0, 0)),
        ],
        out_specs=[
            pl.BlockSpec((1, 1, _TB), lambda i: (i, 0, 0)),
            pl.BlockSpec((1, 1, _TB), lambda i: (i, 0, 0)),
        ],
        out_shape=[
            jax.ShapeDtypeStruct((_NBLK, 1, _TB), jnp.int32),
            jax.ShapeDtypeStruct((_NBLK, 1, _TB), jnp.float32),
        ],
    )(x, W_proj, b_proj, codebook)

    table, lsum = pl.pallas_call(
        _out_table_body,
        grid=(8,),
        in_specs=[
            pl.BlockSpec((_HID, _CB // 8), lambda i: (0, i)),
            pl.BlockSpec((_HID, _IN), lambda i: (0, 0)),
            pl.BlockSpec((_IN,), lambda i: (0,)),
            pl.BlockSpec((_NBLK // 8, 1, _TB), lambda i: (i, 0, 0)),
        ],
        out_specs=[
            pl.BlockSpec((_CB // 8, _IN), lambda i: (i, 0)),
            pl.BlockSpec((1, 1), lambda i: (0, 0)),
        ],
        out_shape=[
            jax.ShapeDtypeStruct((_CB, _IN), jnp.float32),
            jax.ShapeDtypeStruct((1, 1), jnp.float32),
        ],
    )(codebook, W_back, b_back, mv3)

    idx = idx3.reshape(_N)
    out_rows = _make_sc_gather(_N, _IN)(idx, table)

    out = out_rows.reshape(bsz, T, _IN)
    cb_loss = lsum[0, 0] / jnp.float32(_N * _HID)
    att_scores = jnp.zeros((1, 1, 2), dtype=jnp.float32)
    return out, cb_loss, att_scores


# speedup vs baseline: 3.4476x; 1.0034x over previous
import functools

import jax
import jax.numpy as jnp
from jax import lax
from jax.experimental import pallas as pl
from jax.experimental.pallas import tpu as pltpu
from jax.experimental.pallas import tpu_sc as plsc

_HID = 32
_CB = 8192
_IN = 256
_N = 8192
_TB = 512
_NBLK = _N // _TB


_BIG = 2**30
_CHUNK = 2048


def _dist_argmin_body(x_ref, wp_ref, bp_ref, cb_ref, idx_ref, mv_ref,
                      dist_ref):
    h = jnp.dot(x_ref[...], wp_ref[...]) + bp_ref[...][None, :]
    mm = jnp.dot(h, cb_ref[...])
    hnorm = jnp.sum(h * h, axis=1, keepdims=True)
    enorm = jnp.sum(cb_ref[...] * cb_ref[...], axis=0, keepdims=True)
    dist_ref[...] = hnorm - 2.0 * mm + enorm
    dist = dist_ref[...]

    av = ai = mval = None
    for c in range(_CB // _CHUNK):
        dc = dist[:, c * _CHUNK:(c + 1) * _CHUNK]
        mc = jnp.min(dc, axis=1, keepdims=True)
        jc = (jnp.argmin(dc, axis=1).astype(jnp.int32)
              + jnp.int32(c * _CHUNK))[:, None]
        if av is None:
            av, ai, mval = mc, jc, mc
        else:
            avq = av.astype(jnp.bfloat16).astype(jnp.float32)
            upd = mc < avq
            av = jnp.where(upd, mc, avq)
            ai = jnp.where(upd, jc, ai)
            mval = jnp.where(upd, mc, mval)
    idx_ref[0, 0, :] = ai[:, 0]
    mv_ref[0, 0, :] = mval[:, 0]


def _out_table_body(cb_ref, wb_ref, bb_ref, mv_ref, tab_ref, lsum_ref):
    i = pl.program_id(0)
    cb = cb_ref[...]
    tab_ref[...] = lax.dot_general(
        cb, wb_ref[...], (((0,), (0,)), ((), ()))
    ) + bb_ref[...][None, :]
    blk = jnp.sum(mv_ref[...]).reshape(1, 1)

    @pl.when(i == 0)
    def _init():
        lsum_ref[...] = blk

    @pl.when(i > 0)
    def _acc():
        lsum_ref[...] += blk


def _make_sc_gather(n_rows, row_dim, chunk=128):
    info = plsc.get_sparse_core_info()
    nc, ns = info.num_cores, info.num_subcores
    nw = nc * ns
    b_per_w = n_rows // nw
    mesh = plsc.VectorSubcoreMesh(core_axis_name="c", subcore_axis_name="s")

    @functools.partial(
        pl.kernel, mesh=mesh,
        out_type=jax.ShapeDtypeStruct((n_rows, row_dim), jnp.float32),
        scratch_types=[
            pltpu.VMEM((b_per_w,), jnp.int32),
            pltpu.VMEM((b_per_w, row_dim), jnp.float32),
            pltpu.SemaphoreType.DMA,
        ],
    )
    def gather_k(idx_hbm, table_hbm, out_hbm, idx_v, rows_v, sem):
        wid = lax.axis_index("s") * nc + lax.axis_index("c")
        base = wid * b_per_w
        pltpu.sync_copy(idx_hbm.at[pl.ds(base, b_per_w)], idx_v)
        handles = [
            pltpu.async_copy(
                table_hbm.at[idx_v.at[pl.ds(j * chunk, chunk)]],
                rows_v.at[pl.ds(j * chunk, chunk)],
                sem,
            )
            for j in range(b_per_w // chunk)
        ]
        for h in handles:
            h.wait()
        pltpu.sync_copy(rows_v, out_hbm.at[pl.ds(base, b_per_w)])

    return gather_k


def kernel(state, W_proj, b_proj, W_back, b_back, codebook):
    bsz, T, _ = state.shape
    x = state.reshape(_N, _IN)

    idx3, mv3 = pl.pallas_call(
        _dist_argmin_body,
        grid=(_NBLK,),
        in_specs=[
            pl.BlockSpec((_TB, _IN), lambda i: (i, 0)),
            pl.BlockSpec((_IN, _HID), lambda i: (0, 0)),
            pl.BlockSpec((_HID,), lambda i: (0,)),
            pl.BlockSpec((_HID, _CB), lambda i: (0, 0)),
        ],
        out_specs=[
            pl.BlockSpec((1, 1, _TB), lambda i: (i, 0, 0)),
            pl.BlockSpec((1, 1, _TB), lambda i: (i, 0, 0)),
        ],
        out_shape=[
            jax.ShapeDtypeStruct((_NBLK, 1, _TB), jnp.int32),
            jax.ShapeDtypeStruct((_NBLK, 1, _TB), jnp.float32),
        ],
        scratch_shapes=[pltpu.VMEM((_TB, _CB), jnp.float32)],
    )(x, W_proj, b_proj, codebook)

    table, lsum = pl.pallas_call(
        _out_table_body,
        grid=(8,),
        in_specs=[
            pl.BlockSpec((_HID, _CB // 8), lambda i: (0, i)),
            pl.BlockSpec((_HID, _IN), lambda i: (0, 0)),
            pl.BlockSpec((_IN,), lambda i: (0,)),
            pl.BlockSpec((_NBLK // 8, 1, _TB), lambda i: (i, 0, 0)),
        ],
        out_specs=[
            pl.BlockSpec((_CB // 8, _IN), lambda i: (i, 0)),
            pl.BlockSpec((1, 1), lambda i: (0, 0)),
        ],
        out_shape=[
            jax.ShapeDtypeStruct((_CB, _IN), jnp.float32),
            jax.ShapeDtypeStruct((1, 1), jnp.float32),
        ],
    )(codebook, W_back, b_back, mv3)

    idx = idx3.reshape(_N)
    out_rows = _make_sc_gather(_N, _IN)(idx, table)

    out = out_rows.reshape(bsz, T, _IN)
    cb_loss = lsum[0, 0] / jnp.float32(_N * _HID)
    att_scores = jnp.zeros((1, 1, 2), dtype=jnp.float32)
    return out, cb_loss, att_scores


# speedup vs baseline: 3.5746x; 1.0368x over previous
import functools

import jax
import jax.numpy as jnp
from jax import lax
from jax.experimental import pallas as pl
from jax.experimental.pallas import tpu as pltpu
from jax.experimental.pallas import tpu_sc as plsc

_HID = 32
_CB = 8192
_IN = 256
_N = 8192
_TB = 512
_NBLK = _N // _TB


_BIG = 2**30
_CHUNK = 2048


def _dist_argmin_body(x_ref, wp_ref, bp_ref, cb_ref, cbt_ref, wb_ref, bb_ref,
                      idx_ref, tab_ref, lsum_ref, dist_ref):
    i = pl.program_id(0)
    h = jnp.dot(x_ref[...], wp_ref[...]) + bp_ref[...][None, :]
    mm = jnp.dot(h, cb_ref[...])
    hnorm = jnp.sum(h * h, axis=1, keepdims=True)
    enorm = jnp.sum(cb_ref[...] * cb_ref[...], axis=0, keepdims=True)
    dist_ref[...] = hnorm - 2.0 * mm + enorm
    dist = dist_ref[...]

    tab_ref[...] = lax.dot_general(
        cbt_ref[...], wb_ref[...], (((0,), (0,)), ((), ()))
    ) + bb_ref[...][None, :]

    av = ai = mval = None
    for c in range(_CB // _CHUNK):
        dc = dist[:, c * _CHUNK:(c + 1) * _CHUNK]
        mc = jnp.min(dc, axis=1, keepdims=True)
        jc = (jnp.argmin(dc, axis=1).astype(jnp.int32)
              + jnp.int32(c * _CHUNK))[:, None]
        if av is None:
            av, ai, mval = mc, jc, mc
        else:
            avq = av.astype(jnp.bfloat16).astype(jnp.float32)
            upd = mc < avq
            av = jnp.where(upd, mc, avq)
            ai = jnp.where(upd, jc, ai)
            mval = jnp.where(upd, mc, mval)
    idx_ref[0, 0, :] = ai[:, 0]

    blk = jnp.sum(mval).reshape(1, 1)

    @pl.when(i == 0)
    def _init():
        lsum_ref[...] = blk

    @pl.when(i > 0)
    def _acc():
        lsum_ref[...] += blk


def _make_sc_gather(n_rows, row_dim, chunk=128):
    info = plsc.get_sparse_core_info()
    nc, ns = info.num_cores, info.num_subcores
    nw = nc * ns
    b_per_w = n_rows // nw
    mesh = plsc.VectorSubcoreMesh(core_axis_name="c", subcore_axis_name="s")

    @functools.partial(
        pl.kernel, mesh=mesh,
        out_type=jax.ShapeDtypeStruct((n_rows, row_dim), jnp.float32),
        scratch_types=[
            pltpu.VMEM((b_per_w,), jnp.int32),
            pltpu.VMEM((b_per_w, row_dim), jnp.float32),
            pltpu.SemaphoreType.DMA,
        ],
    )
    def gather_k(idx_hbm, table_hbm, out_hbm, idx_v, rows_v, sem):
        wid = lax.axis_index("s") * nc + lax.axis_index("c")
        base = wid * b_per_w
        pltpu.sync_copy(idx_hbm.at[pl.ds(base, b_per_w)], idx_v)
        handles = [
            pltpu.async_copy(
                table_hbm.at[idx_v.at[pl.ds(j * chunk, chunk)]],
                rows_v.at[pl.ds(j * chunk, chunk)],
                sem,
            )
            for j in range(b_per_w // chunk)
        ]
        for h in handles:
            h.wait()
        pltpu.sync_copy(rows_v, out_hbm.at[pl.ds(base, b_per_w)])

    return gather_k


def kernel(state, W_proj, b_proj, W_back, b_back, codebook):
    bsz, T, _ = state.shape
    x = state.reshape(_N, _IN)

    idx3, table, lsum = pl.pallas_call(
        _dist_argmin_body,
        grid=(_NBLK,),
        in_specs=[
            pl.BlockSpec((_TB, _IN), lambda i: (i, 0)),
            pl.BlockSpec((_IN, _HID), lambda i: (0, 0)),
            pl.BlockSpec((_HID,), lambda i: (0,)),
            pl.BlockSpec((_HID, _CB), lambda i: (0, 0)),
            pl.BlockSpec((_HID, _TB), lambda i: (0, i)),
            pl.BlockSpec((_HID, _IN), lambda i: (0, 0)),
            pl.BlockSpec((_IN,), lambda i: (0,)),
        ],
        out_specs=[
            pl.BlockSpec((1, 1, _TB), lambda i: (i, 0, 0)),
            pl.BlockSpec((_TB, _IN), lambda i: (i, 0)),
            pl.BlockSpec((1, 1), lambda i: (0, 0)),
        ],
        out_shape=[
            jax.ShapeDtypeStruct((_NBLK, 1, _TB), jnp.int32),
            jax.ShapeDtypeStruct((_CB, _IN), jnp.float32),
            jax.ShapeDtypeStruct((1, 1), jnp.float32),
        ],
        scratch_shapes=[pltpu.VMEM((_TB, _CB), jnp.float32)],
    )(x, W_proj, b_proj, codebook, codebook, W_back, b_back)

    idx = idx3.reshape(_N)
    out_rows = _make_sc_gather(_N, _IN)(idx, table)

    out = out_rows.reshape(bsz, T, _IN)
    cb_loss = lsum[0, 0] / jnp.float32(_N * _HID)
    att_scores = jnp.zeros((1, 1, 2), dtype=jnp.float32)
    return out, cb_loss, att_scores
